# Initial kernel scaffold; baseline (speedup 1.0000x reference)
import jax
import jax.numpy as jnp
from jax.experimental import pallas as pl


def kernel(x, edge_index, edge_attr, batch, W1_self, W1_nbr, W1_edge, b1, W2_self, W2_nbr, W2_edge, b2):
    raise NotImplementedError("write your pallas kernel here")


# speedup vs baseline: 6.0825x; 6.0825x over previous
import functools

import jax
import jax.numpy as jnp
from jax import lax
from jax.experimental import pallas as pl
from jax.experimental.pallas import tpu as pltpu
from jax.experimental.pallas import tpu_sc as plsc

_N = 100000
_G = 64
_NC = 2
_NS = 16
_NW = _NC * _NS
_C = 1000
_RPT = _N // _NS
_BN = 5000


def _sc_attr_pass(attr, dst, z8):
    e = dst.shape[0]
    per_w = e // _NW
    iters = per_w // _C

    mesh = plsc.VectorSubcoreMesh(
        core_axis_name="c", subcore_axis_name="s",
        num_cores=_NC, num_subcores=_NS)

    @functools.partial(
        pl.kernel,
        out_type=jax.ShapeDtypeStruct((_NC, _N, 8), jnp.float32),
        mesh=mesh,
        scratch_types=[
            pltpu.VMEM((_C,), jnp.int32),
            pltpu.VMEM((_C, 8), jnp.float32),
            pltpu.VMEM_SHARED((_N, 8), jnp.float32),
        ],
        compiler_params=pltpu.CompilerParams(use_tc_tiling_on_sc=False),
    )
    def k(attr_ref, dst_ref, z8_ref, se_out, dst_v, attr_v, se_acc):
        c = lax.axis_index("c")
        s = lax.axis_index("s")
        row0 = s * _RPT
        pltpu.sync_copy(z8_ref, se_acc.at[pl.ds(row0, _RPT)])
        pltpu.sync_copy(z8_ref.at[pl.ds(0, _C)], attr_v)
        plsc.subcore_barrier()

        ebase = (c * _NS + s) * per_w

        def chunk(i, carry):
            base = pl.multiple_of(ebase + i * _C, 8)
            pltpu.sync_copy(dst_ref.at[pl.ds(base, _C)], dst_v)
            pltpu.sync_copy(attr_ref.at[pl.ds(base, _C)],
                            attr_v.at[:, pl.ds(0, 4)])
            pltpu.sync_copy(attr_v, se_acc.at[dst_v], add=True)
            return carry

        lax.fori_loop(0, iters, chunk, 0)
        plsc.subcore_barrier()
        pltpu.sync_copy(se_acc.at[pl.ds(row0, _RPT)],
                        se_out.at[c, pl.ds(row0, _RPT)])

    return k(attr, dst, z8)


def _sc_rows_pass(h, src, dst, z16):
    e = src.shape[0]
    per_w = e // _NW
    iters = per_w // _C

    mesh = plsc.VectorSubcoreMesh(
        core_axis_name="c", subcore_axis_name="s",
        num_cores=_NC, num_subcores=_NS)

    @functools.partial(
        pl.kernel,
        out_type=jax.ShapeDtypeStruct((_NC, _N, 16), jnp.float32),
        mesh=mesh,
        scratch_types=[
            pltpu.VMEM((_C,), jnp.int32),
            pltpu.VMEM((_C,), jnp.int32),
            pltpu.VMEM((_C, 16), jnp.float32),
            pltpu.VMEM_SHARED((_N, 16), jnp.float32),
            pltpu.SemaphoreType.DMA,
        ],
        compiler_params=pltpu.CompilerParams(use_tc_tiling_on_sc=False),
    )
    def k(h_ref, src_ref, dst_ref, z16_ref, sh_out,
          src_v, dst_v, rows_v, sh_acc, sem):
        c = lax.axis_index("c")
        s = lax.axis_index("s")
        row0 = s * _RPT
        pltpu.sync_copy(z16_ref, sh_acc.at[pl.ds(row0, _RPT)])
        plsc.subcore_barrier()

        ebase = (c * _NS + s) * per_w

        def chunk(i, carry):
            base = pl.multiple_of(ebase + i * _C, 8)
            pltpu.sync_copy(src_ref.at[pl.ds(base, _C)], src_v)
            pltpu.sync_copy(dst_ref.at[pl.ds(base, _C)], dst_v)
            pltpu.async_copy(h_ref.at[src_v], rows_v, sem).wait()
            pltpu.sync_copy(rows_v, sh_acc.at[dst_v], add=True)
            return carry

        lax.fori_loop(0, iters, chunk, 0)
        plsc.subcore_barrier()
        pltpu.sync_copy(sh_acc.at[pl.ds(row0, _RPT)],
                        sh_out.at[c, pl.ds(row0, _RPT)])

    return k(h, src, dst, z16)


def _layer1_body(x_ref, sx_ref, se_ref, w1s_ref, w1n_ref, w1e_ref, b1_ref,
                 h_ref):
    xb = x_ref[...]
    sx = sx_ref[0] + sx_ref[1]
    se = (se_ref[0] + se_ref[1])[:, 0:4]
    h = jnp.dot(xb, w1s_ref[...], preferred_element_type=jnp.float32)
    h = h + jnp.dot(sx, w1n_ref[...], preferred_element_type=jnp.float32)
    h = h + jnp.dot(se, w1e_ref[0:4, :], preferred_element_type=jnp.float32)
    h = h + b1_ref[0:1, :]
    h_ref[...] = jnp.maximum(h, 0.0)


def _tc_layer1(x_pad, sx2, se2, w1s, w1n, w1e, b1p):
    return pl.pallas_call(
        _layer1_body,
        grid=(_N // _BN,),
        in_specs=[
            pl.BlockSpec((_BN, 16), lambda i: (i, 0)),
            pl.BlockSpec((_NC, _BN, 16), lambda i: (0, i, 0)),
            pl.BlockSpec((_NC, _BN, 8), lambda i: (0, i, 0)),
            pl.BlockSpec((16, 16), lambda i: (0, 0)),
            pl.BlockSpec((16, 16), lambda i: (0, 0)),
            pl.BlockSpec((8, 16), lambda i: (0, 0)),
            pl.BlockSpec((8, 16), lambda i: (0, 0)),
        ],
        out_specs=pl.BlockSpec((_BN, 16), lambda i: (i, 0)),
        out_shape=jax.ShapeDtypeStruct((_N, 16), jnp.float32),
    )(x_pad, sx2, se2, w1s, w1n, w1e, b1p)


def _layer2_body(h_ref, sh_ref, se_ref, batch_ref, w2_ref, out_ref):
    hb = h_ref[...]
    sh = sh_ref[0] + sh_ref[1]
    se = (se_ref[0] + se_ref[1])[:, 0:4]
    h2 = (jnp.sum(hb * w2_ref[0:1, :], axis=1)
          + jnp.sum(sh * w2_ref[1:2, :], axis=1)
          + jnp.sum(se * w2_ref[2:3, 0:4], axis=1))
    ids = batch_ref[...]
    gids = lax.broadcasted_iota(jnp.int32, (1, _G), 1)
    vals = jnp.where(ids == gids, h2[:, None], -jnp.inf)
    m = jnp.max(vals, axis=0)

    @pl.when(pl.program_id(0) == 0)
    def _():
        out_ref[...] = jnp.full((1, _G), -jnp.inf, jnp.float32)

    out_ref[...] = jnp.maximum(out_ref[...], m[None, :])


def _tc_layer2(h, sh2, se2, batch2d, w2):
    return pl.pallas_call(
        _layer2_body,
        grid=(_N // _BN,),
        in_specs=[
            pl.BlockSpec((_BN, 16), lambda i: (i, 0)),
            pl.BlockSpec((_NC, _BN, 16), lambda i: (0, i, 0)),
            pl.BlockSpec((_NC, _BN, 8), lambda i: (0, i, 0)),
            pl.BlockSpec((_BN, 1), lambda i: (i, 0)),
            pl.BlockSpec((8, 16), lambda i: (0, 0)),
        ],
        out_specs=pl.BlockSpec((1, _G), lambda i: (0, 0)),
        out_shape=jax.ShapeDtypeStruct((1, _G), jnp.float32),
        compiler_params=pltpu.CompilerParams(
            dimension_semantics=("arbitrary",)),
    )(h, sh2, se2, batch2d, w2)


def kernel(x, edge_index, edge_attr, batch, W1_self, W1_nbr, W1_edge, b1,
           W2_self, W2_nbr, W2_edge, b2):
    src = edge_index[0].astype(jnp.int32)
    dst = edge_index[1].astype(jnp.int32)
    x_pad = jnp.pad(x, ((0, 0), (0, 16 - x.shape[1])))
    z16 = jnp.zeros((_RPT, 16), jnp.float32)
    z8 = jnp.zeros((_RPT, 8), jnp.float32)

    sx2 = _sc_rows_pass(x_pad, src, dst, z16)
    se2 = _sc_attr_pass(edge_attr, dst, z8)

    w1s = jnp.zeros((16, 16), jnp.float32).at[:9].set(W1_self)
    w1n = jnp.zeros((16, 16), jnp.float32).at[:9].set(W1_nbr)
    w1e = jnp.zeros((8, 16), jnp.float32).at[:4].set(W1_edge)
    b1p = jnp.broadcast_to(b1[None, :], (8, 16))
    h = _tc_layer1(x_pad, sx2, se2, w1s, w1n, w1e, b1p)

    sh2 = _sc_rows_pass(h, src, dst, z16)

    w2 = (jnp.zeros((8, 16), jnp.float32)
          .at[0, :].set(W2_self[:, 0])
          .at[1, :].set(W2_nbr[:, 0])
          .at[2, :4].set(W2_edge[:, 0]))
    seg = _tc_layer2(h, sh2, se2, batch.reshape(_N, 1).astype(jnp.int32), w2)
    return seg.reshape(_G, 1) + b2[None, :]


# speedup vs baseline: 25.4159x; 4.1785x over previous
import functools

import jax
import jax.numpy as jnp
from jax import lax
from jax.experimental import pallas as pl
from jax.experimental.pallas import tpu as pltpu
from jax.experimental.pallas import tpu_sc as plsc

_N = 100000
_G = 64
_NC = 2
_NS = 16
_NW = _NC * _NS
_C = 1000
_RPT = _N // _NS
_BN = 5000


_EB = 128
_BPC = 8
_CE = _EB * _BPC
_W0 = 12


def _sc_attr_pass(attr_r, dst, z8):
    nblocks = attr_r.shape[0]
    nw_act = _W0 + (_W0 + 1)
    blk_per_w = nblocks // nw_act
    iters = blk_per_w // _BPC
    assert blk_per_w * nw_act == nblocks and iters * _BPC == blk_per_w

    mesh = plsc.VectorSubcoreMesh(
        core_axis_name="c", subcore_axis_name="s",
        num_cores=_NC, num_subcores=_NS)

    @functools.partial(
        pl.kernel,
        out_type=jax.ShapeDtypeStruct((_NC, _N, 8), jnp.float32),
        mesh=mesh,
        scratch_types=[
            pltpu.VMEM((_CE,), jnp.int32),
            pltpu.VMEM((_BPC, 4, _EB), jnp.float32),
            pltpu.VMEM((_CE, 8), jnp.float32),
            pltpu.VMEM_SHARED((_N, 8), jnp.float32),
        ],
        compiler_params=pltpu.CompilerParams(use_tc_tiling_on_sc=False,
                                            needs_layout_passes=False),
    )
    def k(attr_ref, dst_ref, z8_ref, se_out, dst_v, blk_v, rows_v, se_acc):
        c = lax.axis_index("c")
        s = lax.axis_index("s")
        row0 = s * _RPT
        pltpu.sync_copy(z8_ref, se_acc.at[pl.ds(row0, _RPT)])
        pltpu.sync_copy(z8_ref.at[pl.ds(0, _CE)], rows_v)
        plsc.subcore_barrier()

        rank = s + c * _W0
        nact = _W0 + c
        lane = lax.iota(jnp.int32, 16)

        def chunk(i, carry):
            blk0 = rank * blk_per_w + i * _BPC
            e0 = pl.multiple_of(blk0 * _EB, 8)
            pltpu.sync_copy(attr_ref.at[pl.ds(blk0, _BPC)], blk_v)
            pltpu.sync_copy(dst_ref.at[pl.ds(e0, _CE)], dst_v)
            for b in range(_BPC):
                for cc in range(4):
                    col = jnp.full((16,), cc, jnp.int32)
                    for v in range(_EB // 16):
                        vec = blk_v[b, cc, pl.ds(16 * v, 16)]
                        ridx = lane + (b * _EB + 16 * v)
                        plsc.store_scatter(rows_v, [ridx, col], vec)
            pltpu.sync_copy(rows_v, se_acc.at[dst_v], add=True)
            return carry

        @pl.when(s < nact)
        def _():
            lax.fori_loop(0, iters, chunk, 0)

        plsc.subcore_barrier()
        pltpu.sync_copy(se_acc.at[pl.ds(row0, _RPT)],
                        se_out.at[c, pl.ds(row0, _RPT)])

    return k(attr_r, dst, z8)


def _sc_rows_pass(h, src, dst, z16):
    e = src.shape[0]
    per_w = e // _NW
    iters = per_w // _C

    mesh = plsc.VectorSubcoreMesh(
        core_axis_name="c", subcore_axis_name="s",
        num_cores=_NC, num_subcores=_NS)

    @functools.partial(
        pl.kernel,
        out_type=jax.ShapeDtypeStruct((_NC, _N, 16), jnp.float32),
        mesh=mesh,
        scratch_types=[
            pltpu.VMEM((_C,), jnp.int32),
            pltpu.VMEM((_C,), jnp.int32),
            pltpu.VMEM((_C, 16), jnp.float32),
            pltpu.VMEM_SHARED((_N, 16), jnp.float32),
            pltpu.SemaphoreType.DMA,
        ],
        compiler_params=pltpu.CompilerParams(use_tc_tiling_on_sc=False),
    )
    def k(h_ref, src_ref, dst_ref, z16_ref, sh_out,
          src_v, dst_v, rows_v, sh_acc, sem):
        c = lax.axis_index("c")
        s = lax.axis_index("s")
        row0 = s * _RPT
        pltpu.sync_copy(z16_ref, sh_acc.at[pl.ds(row0, _RPT)])
        plsc.subcore_barrier()

        ebase = (c * _NS + s) * per_w

        def chunk(i, carry):
            base = pl.multiple_of(ebase + i * _C, 8)
            pltpu.sync_copy(src_ref.at[pl.ds(base, _C)], src_v)
            pltpu.sync_copy(dst_ref.at[pl.ds(base, _C)], dst_v)
            pltpu.async_copy(h_ref.at[src_v], rows_v, sem).wait()
            pltpu.sync_copy(rows_v, sh_acc.at[dst_v], add=True)
            return carry

        lax.fori_loop(0, iters, chunk, 0)
        plsc.subcore_barrier()
        pltpu.sync_copy(sh_acc.at[pl.ds(row0, _RPT)],
                        sh_out.at[c, pl.ds(row0, _RPT)])

    return k(h, src, dst, z16)


def _layer1_body(x_ref, sx_ref, se_ref, w1s_ref, w1n_ref, w1e_ref, b1_ref,
                 h_ref):
    xb = x_ref[...]
    sx = sx_ref[0] + sx_ref[1]
    se = (se_ref[0] + se_ref[1])[:, 0:4]
    h = jnp.dot(xb, w1s_ref[...], preferred_element_type=jnp.float32)
    h = h + jnp.dot(sx, w1n_ref[...], preferred_element_type=jnp.float32)
    h = h + jnp.dot(se, w1e_ref[0:4, :], preferred_element_type=jnp.float32)
    h = h + b1_ref[0:1, :]
    h_ref[...] = jnp.maximum(h, 0.0)


def _tc_layer1(x_pad, sx2, se2, w1s, w1n, w1e, b1p):
    return pl.pallas_call(
        _layer1_body,
        grid=(_N // _BN,),
        in_specs=[
            pl.BlockSpec((_BN, 16), lambda i: (i, 0)),
            pl.BlockSpec((_NC, _BN, 16), lambda i: (0, i, 0)),
            pl.BlockSpec((_NC, _BN, 8), lambda i: (0, i, 0)),
            pl.BlockSpec((16, 16), lambda i: (0, 0)),
            pl.BlockSpec((16, 16), lambda i: (0, 0)),
            pl.BlockSpec((8, 16), lambda i: (0, 0)),
            pl.BlockSpec((8, 16), lambda i: (0, 0)),
        ],
        out_specs=pl.BlockSpec((_BN, 16), lambda i: (i, 0)),
        out_shape=jax.ShapeDtypeStruct((_N, 16), jnp.float32),
    )(x_pad, sx2, se2, w1s, w1n, w1e, b1p)


def _layer2_body(h_ref, sh_ref, se_ref, batch_ref, w2_ref, out_ref):
    hb = h_ref[...]
    sh = sh_ref[0] + sh_ref[1]
    se = (se_ref[0] + se_ref[1])[:, 0:4]
    h2 = (jnp.sum(hb * w2_ref[0:1, :], axis=1)
          + jnp.sum(sh * w2_ref[1:2, :], axis=1)
          + jnp.sum(se * w2_ref[2:3, 0:4], axis=1))
    ids = batch_ref[...]
    gids = lax.broadcasted_iota(jnp.int32, (1, _G), 1)
    vals = jnp.where(ids == gids, h2[:, None], -jnp.inf)
    m = jnp.max(vals, axis=0)

    @pl.when(pl.program_id(0) == 0)
    def _():
        out_ref[...] = jnp.full((1, _G), -jnp.inf, jnp.float32)

    out_ref[...] = jnp.maximum(out_ref[...], m[None, :])


def _tc_layer2(h, sh2, se2, batch2d, w2):
    return pl.pallas_call(
        _layer2_body,
        grid=(_N // _BN,),
        in_specs=[
            pl.BlockSpec((_BN, 16), lambda i: (i, 0)),
            pl.BlockSpec((_NC, _BN, 16), lambda i: (0, i, 0)),
            pl.BlockSpec((_NC, _BN, 8), lambda i: (0, i, 0)),
            pl.BlockSpec((_BN, 1), lambda i: (i, 0)),
            pl.BlockSpec((8, 16), lambda i: (0, 0)),
        ],
        out_specs=pl.BlockSpec((1, _G), lambda i: (0, 0)),
        out_shape=jax.ShapeDtypeStruct((1, _G), jnp.float32),
        compiler_params=pltpu.CompilerParams(
            dimension_semantics=("arbitrary",)),
    )(h, sh2, se2, batch2d, w2)


def kernel(x, edge_index, edge_attr, batch, W1_self, W1_nbr, W1_edge, b1,
           W2_self, W2_nbr, W2_edge, b2):
    src = edge_index[0].astype(jnp.int32)
    dst = edge_index[1].astype(jnp.int32)
    x_pad = jnp.pad(x, ((0, 0), (0, 16 - x.shape[1])))
    z16 = jnp.zeros((_RPT, 16), jnp.float32)
    z8 = jnp.zeros((_RPT, 8), jnp.float32)

    e = edge_attr.shape[0]
    attr_r = jnp.transpose(edge_attr.reshape(e // _EB, _EB, 4), (0, 2, 1))
    sx2 = _sc_rows_pass(x_pad, src, dst, z16)
    se2 = _sc_attr_pass(attr_r, dst, z8)

    w1s = jnp.zeros((16, 16), jnp.float32).at[:9].set(W1_self)
    w1n = jnp.zeros((16, 16), jnp.float32).at[:9].set(W1_nbr)
    w1e = jnp.zeros((8, 16), jnp.float32).at[:4].set(W1_edge)
    b1p = jnp.broadcast_to(b1[None, :], (8, 16))
    h = _tc_layer1(x_pad, sx2, se2, w1s, w1n, w1e, b1p)

    sh2 = _sc_rows_pass(h, src, dst, z16)

    w2 = (jnp.zeros((8, 16), jnp.float32)
          .at[0, :].set(W2_self[:, 0])
          .at[1, :].set(W2_nbr[:, 0])
          .at[2, :4].set(W2_edge[:, 0]))
    seg = _tc_layer2(h, sh2, se2, batch.reshape(_N, 1).astype(jnp.int32), w2)
    return seg.reshape(_G, 1) + b2[None, :]


# speedup vs baseline: 26.0358x; 1.0244x over previous
import functools

import jax
import jax.numpy as jnp
from jax import lax
from jax.experimental import pallas as pl
from jax.experimental.pallas import tpu as pltpu
from jax.experimental.pallas import tpu_sc as plsc

_N = 100000
_G = 64
_NC = 2
_NS = 16
_NW = _NC * _NS
_C = 1000
_RPT = _N // _NS
_BN = 5000


_EB = 128
_BPC = 8
_CE = _EB * _BPC
_W0 = 12


def _sc_xattr_pass(x_pad, src, dst, attr_r, z16):
    nblocks = attr_r.shape[0]
    nw_act = _W0 + (_W0 + 1)
    blk_per_w = nblocks // nw_act
    iters = blk_per_w // _BPC
    assert blk_per_w * nw_act == nblocks and iters * _BPC == blk_per_w

    mesh = plsc.VectorSubcoreMesh(
        core_axis_name="c", subcore_axis_name="s",
        num_cores=_NC, num_subcores=_NS)

    @functools.partial(
        pl.kernel,
        out_type=jax.ShapeDtypeStruct((_NC, _N, 16), jnp.float32),
        mesh=mesh,
        scratch_types=[
            pltpu.VMEM((_CE,), jnp.int32),
            pltpu.VMEM((_CE,), jnp.int32),
            pltpu.VMEM((_BPC, 4, _EB), jnp.float32),
            pltpu.VMEM((_CE, 16), jnp.float32),
            pltpu.VMEM_SHARED((_N, 16), jnp.float32),
            pltpu.SemaphoreType.DMA,
        ],
        compiler_params=pltpu.CompilerParams(use_tc_tiling_on_sc=False,
                                             needs_layout_passes=False),
    )
    def k(x_ref, src_ref, dst_ref, attr_ref, z16_ref, s_out,
          src_v, dst_v, blk_v, rows_v, s_acc, sem):
        c = lax.axis_index("c")
        s = lax.axis_index("s")
        row0 = s * _RPT
        pltpu.sync_copy(z16_ref, s_acc.at[pl.ds(row0, _RPT)])
        plsc.subcore_barrier()

        rank = s + c * _W0
        nact = _W0 + c
        lane = lax.iota(jnp.int32, 16)

        def chunk(i, carry):
            blk0 = rank * blk_per_w + i * _BPC
            e0 = pl.multiple_of(blk0 * _EB, 8)
            pltpu.sync_copy(src_ref.at[pl.ds(e0, _CE)], src_v)
            pltpu.sync_copy(dst_ref.at[pl.ds(e0, _CE)], dst_v)
            pltpu.sync_copy(attr_ref.at[pl.ds(blk0, _BPC)], blk_v)
            pltpu.async_copy(x_ref.at[src_v], rows_v, sem).wait()
            for b in range(_BPC):
                for cc in range(4):
                    col = jnp.full((16,), 9 + cc, jnp.int32)
                    for v in range(_EB // 16):
                        vec = blk_v[b, cc, pl.ds(16 * v, 16)]
                        ridx = lane + (b * _EB + 16 * v)
                        plsc.store_scatter(rows_v, [ridx, col], vec)
            pltpu.sync_copy(rows_v, s_acc.at[dst_v], add=True)
            return carry

        @pl.when(s < nact)
        def _():
            lax.fori_loop(0, iters, chunk, 0)

        plsc.subcore_barrier()
        pltpu.sync_copy(s_acc.at[pl.ds(row0, _RPT)],
                        s_out.at[c, pl.ds(row0, _RPT)])

    return k(x_pad, src, dst, attr_r, z16)


def _sc_rows_pass(h, src, dst, z16):
    e = src.shape[0]
    per_w = e // _NW
    iters = per_w // _C

    mesh = plsc.VectorSubcoreMesh(
        core_axis_name="c", subcore_axis_name="s",
        num_cores=_NC, num_subcores=_NS)

    @functools.partial(
        pl.kernel,
        out_type=jax.ShapeDtypeStruct((_NC, _N, 16), jnp.float32),
        mesh=mesh,
        scratch_types=[
            pltpu.VMEM((_C,), jnp.int32),
            pltpu.VMEM((_C,), jnp.int32),
            pltpu.VMEM((_C, 16), jnp.float32),
            pltpu.VMEM_SHARED((_N, 16), jnp.float32),
            pltpu.SemaphoreType.DMA,
        ],
        compiler_params=pltpu.CompilerParams(use_tc_tiling_on_sc=False),
    )
    def k(h_ref, src_ref, dst_ref, z16_ref, sh_out,
          src_v, dst_v, rows_v, sh_acc, sem):
        c = lax.axis_index("c")
        s = lax.axis_index("s")
        row0 = s * _RPT
        pltpu.sync_copy(z16_ref, sh_acc.at[pl.ds(row0, _RPT)])
        plsc.subcore_barrier()

        ebase = (c * _NS + s) * per_w

        def chunk(i, carry):
            base = pl.multiple_of(ebase + i * _C, 8)
            pltpu.sync_copy(src_ref.at[pl.ds(base, _C)], src_v)
            pltpu.sync_copy(dst_ref.at[pl.ds(base, _C)], dst_v)
            pltpu.async_copy(h_ref.at[src_v], rows_v, sem).wait()
            pltpu.sync_copy(rows_v, sh_acc.at[dst_v], add=True)
            return carry

        lax.fori_loop(0, iters, chunk, 0)
        plsc.subcore_barrier()
        pltpu.sync_copy(sh_acc.at[pl.ds(row0, _RPT)],
                        sh_out.at[c, pl.ds(row0, _RPT)])

    return k(h, src, dst, z16)


def _layer1_body(x_ref, sx_ref, w1s_ref, w1n_ref, w1e_ref, b1_ref,
                 h_ref):
    xb = x_ref[...]
    sx = sx_ref[0] + sx_ref[1]
    se = sx[:, 9:13]
    h = jnp.dot(xb, w1s_ref[...], preferred_element_type=jnp.float32)
    h = h + jnp.dot(sx, w1n_ref[...], preferred_element_type=jnp.float32)
    h = h + jnp.dot(se, w1e_ref[0:4, :], preferred_element_type=jnp.float32)
    h = h + b1_ref[0:1, :]
    h_ref[...] = jnp.maximum(h, 0.0)


def _tc_layer1(x_pad, sx2, w1s, w1n, w1e, b1p):
    return pl.pallas_call(
        _layer1_body,
        grid=(_N // _BN,),
        in_specs=[
            pl.BlockSpec((_BN, 16), lambda i: (i, 0)),
            pl.BlockSpec((_NC, _BN, 16), lambda i: (0, i, 0)),
            pl.BlockSpec((16, 16), lambda i: (0, 0)),
            pl.BlockSpec((16, 16), lambda i: (0, 0)),
            pl.BlockSpec((8, 16), lambda i: (0, 0)),
            pl.BlockSpec((8, 16), lambda i: (0, 0)),
        ],
        out_specs=pl.BlockSpec((_BN, 16), lambda i: (i, 0)),
        out_shape=jax.ShapeDtypeStruct((_N, 16), jnp.float32),
    )(x_pad, sx2, w1s, w1n, w1e, b1p)


def _layer2_body(h_ref, sh_ref, sx_ref, batch_ref, w2_ref, out_ref):
    hb = h_ref[...]
    sh = sh_ref[0] + sh_ref[1]
    se = (sx_ref[0] + sx_ref[1])[:, 9:13]
    h2 = (jnp.sum(hb * w2_ref[0:1, :], axis=1)
          + jnp.sum(sh * w2_ref[1:2, :], axis=1)
          + jnp.sum(se * w2_ref[2:3, 0:4], axis=1))
    ids = batch_ref[...]
    gids = lax.broadcasted_iota(jnp.int32, (1, _G), 1)
    vals = jnp.where(ids == gids, h2[:, None], -jnp.inf)
    m = jnp.max(vals, axis=0)

    @pl.when(pl.program_id(0) == 0)
    def _():
        out_ref[...] = jnp.full((1, _G), -jnp.inf, jnp.float32)

    out_ref[...] = jnp.maximum(out_ref[...], m[None, :])


def _tc_layer2(h, sh2, sx2, batch2d, w2):
    return pl.pallas_call(
        _layer2_body,
        grid=(_N // _BN,),
        in_specs=[
            pl.BlockSpec((_BN, 16), lambda i: (i, 0)),
            pl.BlockSpec((_NC, _BN, 16), lambda i: (0, i, 0)),
            pl.BlockSpec((_NC, _BN, 16), lambda i: (0, i, 0)),
            pl.BlockSpec((_BN, 1), lambda i: (i, 0)),
            pl.BlockSpec((8, 16), lambda i: (0, 0)),
        ],
        out_specs=pl.BlockSpec((1, _G), lambda i: (0, 0)),
        out_shape=jax.ShapeDtypeStruct((1, _G), jnp.float32),
        compiler_params=pltpu.CompilerParams(
            dimension_semantics=("arbitrary",)),
    )(h, sh2, sx2, batch2d, w2)


def kernel(x, edge_index, edge_attr, batch, W1_self, W1_nbr, W1_edge, b1,
           W2_self, W2_nbr, W2_edge, b2):
    src = edge_index[0].astype(jnp.int32)
    dst = edge_index[1].astype(jnp.int32)
    x_pad = jnp.pad(x, ((0, 0), (0, 16 - x.shape[1])))
    z16 = jnp.zeros((_RPT, 16), jnp.float32)

    e = edge_attr.shape[0]
    attr_r = jnp.transpose(edge_attr.reshape(e // _EB, _EB, 4), (0, 2, 1))
    sx2 = _sc_xattr_pass(x_pad, src, dst, attr_r, z16)

    w1s = jnp.zeros((16, 16), jnp.float32).at[:9].set(W1_self)
    w1n = jnp.zeros((16, 16), jnp.float32).at[:9].set(W1_nbr)
    w1e = jnp.zeros((8, 16), jnp.float32).at[:4].set(W1_edge)
    b1p = jnp.broadcast_to(b1[None, :], (8, 16))
    h = _tc_layer1(x_pad, sx2, w1s, w1n, w1e, b1p)

    sh2 = _sc_rows_pass(h, src, dst, z16)

    w2 = (jnp.zeros((8, 16), jnp.float32)
          .at[0, :].set(W2_self[:, 0])
          .at[1, :].set(W2_nbr[:, 0])
          .at[2, :4].set(W2_edge[:, 0]))
    seg = _tc_layer2(h, sh2, sx2, batch.reshape(_N, 1).astype(jnp.int32), w2)
    return seg.reshape(_G, 1) + b2[None, :]
